# baseline (speedup 1.0000x reference)
import jax
import jax.numpy as jnp
from jax.experimental import pallas as pl
from jax.experimental.pallas import tpu as pltpu

_BN = 32
_TB = 256


def _body(idx_ref, w_ref, x_ref, o_ref):
    g = pl.program_id(0)
    for j in range(_BN):
        s = idx_ref[g * _BN + j]
        w = w_ref[s].astype(jnp.bfloat16)
        x = x_ref[j].astype(jnp.bfloat16)
        o_ref[j] = jax.lax.dot_general(
            w, x,
            dimension_numbers=(((1,), (0,)), ((), ())),
            preferred_element_type=jnp.float32,
        )


def kernel(X, subject_idx, W):
    N, C, T = X.shape
    S = W.shape[0]

    grid_spec = pltpu.PrefetchScalarGridSpec(
        num_scalar_prefetch=1,
        grid=(N // _BN, T // _TB),
        in_specs=[
            pl.BlockSpec((S, C, C), lambda n, t, idx: (0, 0, 0)),
            pl.BlockSpec((_BN, C, _TB), lambda n, t, idx: (n, 0, t)),
        ],
        out_specs=pl.BlockSpec((_BN, C, _TB), lambda n, t, idx: (n, 0, t)),
    )
    return pl.pallas_call(
        _body,
        grid_spec=grid_spec,
        out_shape=jax.ShapeDtypeStruct((N, C, T), jnp.float32),
        compiler_params=pltpu.CompilerParams(
            vmem_limit_bytes=60 * 1024 * 1024,
        ),
    )(subject_idx, W, X)


# speedup vs baseline: 1.0007x; 1.0007x over previous
import jax
import jax.numpy as jnp
from jax.experimental import pallas as pl
from jax.experimental.pallas import tpu as pltpu

_BN = 16
_TB = 512


def _body(idx_ref, w_ref, x_ref, o_ref):
    g = pl.program_id(0)
    for j in range(_BN):
        s = idx_ref[g * _BN + j]
        w = w_ref[s].astype(jnp.bfloat16)
        x = x_ref[j].astype(jnp.bfloat16)
        o_ref[j] = jax.lax.dot_general(
            w, x,
            dimension_numbers=(((1,), (0,)), ((), ())),
            preferred_element_type=jnp.float32,
        )


def kernel(X, subject_idx, W):
    N, C, T = X.shape
    S = W.shape[0]

    grid_spec = pltpu.PrefetchScalarGridSpec(
        num_scalar_prefetch=1,
        grid=(N // _BN, T // _TB),
        in_specs=[
            pl.BlockSpec((S, C, C), lambda n, t, idx: (0, 0, 0)),
            pl.BlockSpec((_BN, C, _TB), lambda n, t, idx: (n, 0, t)),
        ],
        out_specs=pl.BlockSpec((_BN, C, _TB), lambda n, t, idx: (n, 0, t)),
    )
    return pl.pallas_call(
        _body,
        grid_spec=grid_spec,
        out_shape=jax.ShapeDtypeStruct((N, C, T), jnp.float32),
        compiler_params=pltpu.CompilerParams(
            vmem_limit_bytes=60 * 1024 * 1024,
            dimension_semantics=("parallel", "arbitrary"),
        ),
    )(subject_idx, W, X)
